# Initial kernel scaffold; baseline (speedup 1.0000x reference)
import jax
import jax.numpy as jnp
from jax.experimental import pallas as pl


def kernel(noisy_coords, noisy_types, z, t, edge_index, batch, lattice, offsets, atom_embed, Wz, bz, Wt, bt, Wphi1, bphi1, Wphi2, bphi2, Wfil, bfil, U, V, Wu1, bu1, Wu2, bu2, Wc1, bc1, Wc2, bc2, Wy1, by1, Wy2, by2):
    raise NotImplementedError("write your pallas kernel here")


# speedup vs baseline: 15.7708x; 15.7708x over previous
import functools
import math

import jax
import jax.numpy as jnp
from jax import lax
from jax.experimental import pallas as pl
from jax.experimental.pallas import tpu as pltpu
from jax.experimental.pallas import tpu_sc as plsc

N = 10000
E = 160000
H = 128
R = 64
L = 4
LAT = 128
TD = 64
A = 101
G = 64
CUTOFF = 6.0
GAMMA = R / CUTOFF

NB = 1000
EB = 2000
CH = 128
NCHUNK = E // CH
FC = 200
NFCHUNK = N // FC

def _sc_mesh():
    return plsc.VectorSubcoreMesh(core_axis_name="c", subcore_axis_name="s")


def _silu(x):
    return x * jax.nn.sigmoid(x)


def _dot(a, b):
    return jnp.dot(a, b, preferred_element_type=jnp.float32)


def _gather_pairs(t1, i1, t2, i2):
    D = t1.shape[1]

    @functools.partial(
        pl.kernel,
        mesh=_sc_mesh(),
        out_type=[jax.ShapeDtypeStruct((E, D), jnp.float32),
                  jax.ShapeDtypeStruct((E, D), jnp.float32)],
        scratch_types=[pltpu.VMEM((CH,), jnp.int32),
                       pltpu.VMEM((CH,), jnp.int32),
                       pltpu.VMEM((CH, D), jnp.float32),
                       pltpu.VMEM((CH, D), jnp.float32),
                       pltpu.SemaphoreType.DMA,
                       pltpu.SemaphoreType.DMA],
    )
    def k(t1_hbm, i1_hbm, t2_hbm, i2_hbm, o1_hbm, o2_hbm,
          ix1, ix2, r1, r2, sem1, sem2):
        wid = lax.axis_index("s") * 2 + lax.axis_index("c")
        nfull = NCHUNK // 32 + 1

        @pl.loop(0, nfull)
        def _(j):
            cid = j * 32 + wid

            @pl.when(cid < NCHUNK)
            def _():
                base = cid * CH
                pltpu.sync_copy(i1_hbm.at[pl.ds(base, CH)], ix1)
                pltpu.sync_copy(i2_hbm.at[pl.ds(base, CH)], ix2)
                c1 = pltpu.async_copy(t1_hbm.at[ix1], r1, sem1)
                c2 = pltpu.async_copy(t2_hbm.at[ix2], r2, sem2)
                c1.wait()
                c2.wait()
                pltpu.sync_copy(r1, o1_hbm.at[pl.ds(base, CH)])
                pltpu.sync_copy(r2, o2_hbm.at[pl.ds(base, CH)])

    return k(t1, i1, t2, i2)


def _scatter_sums(M, dst, zrows):
    @functools.partial(
        pl.kernel,
        mesh=_sc_mesh(),
        out_type=jax.ShapeDtypeStruct((N, 4 * H), jnp.float32),
        scratch_types=[pltpu.VMEM((CH,), jnp.int32),
                       pltpu.VMEM((CH, H), jnp.float32),
                       pltpu.VMEM((FC, H), jnp.float32),
                       pltpu.VMEM_SHARED((N, H), jnp.float32)],
    )
    def k(m_hbm, ix_hbm, z_hbm, out_hbm, idx_v, m_v, fbuf, acc):
        core = lax.axis_index("c")
        sub = lax.axis_index("s")
        nec = NCHUNK // 16 + 1
        nfc = NFCHUNK // 16 + 1

        for p in range(2):
            slab = p * 2 + core

            pltpu.sync_copy(z_hbm, fbuf)

            @pl.loop(0, nfc)
            def _(j):
                fc = j * 16 + sub

                @pl.when(fc < NFCHUNK)
                def _():
                    pltpu.sync_copy(fbuf, acc.at[pl.ds(fc * FC, FC)])

            plsc.subcore_barrier()

            @pl.loop(0, nec)
            def _(j):
                cid = j * 16 + sub

                @pl.when(cid < NCHUNK)
                def _():
                    base = cid * CH
                    pltpu.sync_copy(ix_hbm.at[pl.ds(base, CH)], idx_v)
                    pltpu.sync_copy(
                        m_hbm.at[pl.ds(base, CH), pl.ds(slab * H, H)], m_v)
                    pltpu.sync_copy(m_v, acc.at[idx_v], add=True)

            plsc.subcore_barrier()

            @pl.loop(0, nfc)
            def _(j):
                fc = j * 16 + sub

                @pl.when(fc < NFCHUNK)
                def _():
                    pltpu.sync_copy(acc.at[pl.ds(fc * FC, FC)], fbuf)
                    pltpu.sync_copy(
                        fbuf, out_hbm.at[pl.ds(fc * FC, FC),
                                         pl.ds(slab * H, H)])

            plsc.subcore_barrier()

    return k(M, dst, zrows)


def _cond_body(z_ref, t_ref, wz_ref, bz_ref, wt_ref, bt_ref, o_ref):
    t = t_ref[...].astype(jnp.float32)
    half = TD // 2
    j = lax.broadcasted_iota(jnp.int32, (1, half), 1).astype(jnp.float32)
    freqs = jnp.exp(-math.log(10000.0) * j / (half - 1))
    emb = t * freqs
    temb = jnp.concatenate([jnp.sin(emb), jnp.cos(emb)], axis=1)
    o_ref[...] = (_dot(z_ref[...], wz_ref[...]) + bz_ref[...]
                  + _dot(temb, wt_ref[...]) + bt_ref[...])


def _cond_graphs(z, t, Wz, bz, Wt, bt):
    return pl.pallas_call(
        _cond_body,
        out_shape=jax.ShapeDtypeStruct((G, H), jnp.float32),
    )(z, t, Wz, bz, Wt, bt)


def _init_body(ty_ref, b_ref, ae_ref, cg_ref, s0_ref, cond_ref):
    ty = ty_ref[...]
    oh = (ty == lax.broadcasted_iota(jnp.int32, (NB, A), 1))
    s0_ref[...] = _dot(oh.astype(jnp.float32), ae_ref[...])
    b = b_ref[...]
    ohb = (b == lax.broadcasted_iota(jnp.int32, (NB, G), 1))
    cond_ref[...] = _dot(ohb.astype(jnp.float32), cg_ref[...])


def _node_init(types2, batch2, atom_embed, condG):
    full = lambda s: pl.BlockSpec(s, lambda i: (0, 0))
    return pl.pallas_call(
        _init_body,
        grid=(N // NB,),
        in_specs=[pl.BlockSpec((NB, 1), lambda i: (i, 0)),
                  pl.BlockSpec((NB, 1), lambda i: (i, 0)),
                  full((A, H)), full((G, H))],
        out_specs=[pl.BlockSpec((NB, H), lambda i: (i, 0)),
                   pl.BlockSpec((NB, H), lambda i: (i, 0))],
        out_shape=[jax.ShapeDtypeStruct((N, H), jnp.float32),
                   jax.ShapeDtypeStruct((N, H), jnp.float32)],
    )(types2, batch2, atom_embed, condG)


def _table_body(xy_ref, b_ref, lat_ref, t_ref):
    b = b_ref[...]
    ohb = (b == lax.broadcasted_iota(jnp.int32, (NB, G), 1))
    lrows = _dot(ohb.astype(jnp.float32), lat_ref[...])
    t_ref[...] = jnp.concatenate(
        [xy_ref[...], lrows, jnp.zeros((NB, 116), jnp.float32)], axis=1)


def _node_table(coords, batch2, latflat):
    return pl.pallas_call(
        _table_body,
        grid=(N // NB,),
        in_specs=[pl.BlockSpec((NB, 3), lambda i: (i, 0)),
                  pl.BlockSpec((NB, 1), lambda i: (i, 0)),
                  pl.BlockSpec((G, 9), lambda i: (0, 0))],
        out_specs=pl.BlockSpec((NB, 128), lambda i: (i, 0)),
        out_shape=jax.ShapeDtypeStruct((N, 128), jnp.float32),
    )(coords, batch2, latflat)


def _geo_body(ts_ref, td_ref, off_ref, geo_ref):
    ts = ts_ref[...]
    td = td_ref[...]
    frac = td[:, 0:3] - ts[:, 0:3] + off_ref[...]
    lf = ts[:, 3:12]
    cols = []
    for jx in range(3):
        cj = (frac[:, 0:1] * lf[:, jx:jx + 1]
              + frac[:, 1:2] * lf[:, 3 + jx:4 + jx]
              + frac[:, 2:3] * lf[:, 6 + jx:7 + jx])
        cols.append(cj)
    cart = jnp.concatenate(cols, axis=1)
    nrm = jnp.sqrt(jnp.sum(cart * cart, axis=1, keepdims=True))
    d = jnp.clip(nrm, 1e-8, None)
    unit = cart / jnp.maximum(nrm, 1e-12)
    geo_ref[...] = jnp.concatenate(
        [d, unit, jnp.zeros((EB, 4), jnp.float32)], axis=1)


def _edge_geometry(tsrc, tdst, offsets):
    return pl.pallas_call(
        _geo_body,
        grid=(E // EB,),
        in_specs=[pl.BlockSpec((EB, 128), lambda i: (i, 0)),
                  pl.BlockSpec((EB, 128), lambda i: (i, 0)),
                  pl.BlockSpec((EB, 3), lambda i: (i, 0))],
        out_specs=pl.BlockSpec((EB, 8), lambda i: (i, 0)),
        out_shape=jax.ShapeDtypeStruct((E, 8), jnp.float32),
    )(tsrc, tdst, offsets)


def _phi_body(s_ref, c_ref, w1_ref, b1_ref, w2_ref, b2_ref,
              snew_ref, phi_ref):
    s1 = s_ref[...] + c_ref[...]
    snew_ref[...] = s1
    h = _silu(_dot(s1, w1_ref[...]) + b1_ref[...])
    phi_ref[...] = _dot(h, w2_ref[...]) + b2_ref[...]


def _phi_layer(s, cond, W1, b1, W2, b2):
    full = lambda s_: pl.BlockSpec(s_, lambda i: (0, 0))
    return pl.pallas_call(
        _phi_body,
        grid=(N // NB,),
        in_specs=[pl.BlockSpec((NB, H), lambda i: (i, 0)),
                  pl.BlockSpec((NB, H), lambda i: (i, 0)),
                  full((H, H)), full((1, H)), full((H, 3 * H)),
                  full((1, 3 * H))],
        out_specs=[pl.BlockSpec((NB, H), lambda i: (i, 0)),
                   pl.BlockSpec((NB, 3 * H), lambda i: (i, 0))],
        out_shape=[jax.ShapeDtypeStruct((N, H), jnp.float32),
                   jax.ShapeDtypeStruct((N, 3 * H), jnp.float32)],
    )(s, cond, W1, b1, W2, b2)


def _edge_body(pl_ref, vl_ref, geo_ref, wf_ref, bf_ref, m_ref):
    g = geo_ref[...]
    d = g[:, 0:1]
    centers = lax.broadcasted_iota(jnp.int32, (1, R), 1).astype(
        jnp.float32) * (CUTOFF / (R - 1))
    erbf = jnp.exp(-GAMMA * (d - centers) ** 2)
    filt = _dot(erbf, wf_ref[...]) + bf_ref[...]
    m = pl_ref[...] * filt
    ds = m[:, 0:H]
    dvg = m[:, H:2 * H]
    dvd = m[:, 2 * H:3 * H]
    vl = vl_ref[...]
    parts = [ds]
    for c in range(3):
        parts.append(dvg * vl[:, c * H:(c + 1) * H] + dvd * g[:, 1 + c:2 + c])
    m_ref[...] = jnp.concatenate(parts, axis=1)


def _edge_messages(philist, vlist, geo, Wf, bf):
    full = lambda s_: pl.BlockSpec(s_, lambda i: (0, 0))
    return pl.pallas_call(
        _edge_body,
        grid=(E // EB,),
        in_specs=[pl.BlockSpec((EB, 3 * H), lambda i: (i, 0)),
                  pl.BlockSpec((EB, 3 * H), lambda i: (i, 0)),
                  pl.BlockSpec((EB, 8), lambda i: (i, 0)),
                  full((R, 3 * H)), full((1, 3 * H))],
        out_specs=pl.BlockSpec((EB, 4 * H), lambda i: (i, 0)),
        out_shape=jax.ShapeDtypeStruct((E, 4 * H), jnp.float32),
    )(philist, vlist, geo, Wf, bf)


def _update_body(s_ref, dss_ref, v_ref, dvs_ref, u_ref, vv_ref,
                 w1s_ref, w1n_ref, b1_ref, w2_ref, b2_ref,
                 s2_ref, v2_ref):
    s1 = s_ref[...] + dss_ref[...]
    v1 = v_ref[...] + dvs_ref[...]
    uu = u_ref[...]
    ww = vv_ref[...]
    uv = []
    vv = []
    for c in range(3):
        vc = v1[:, c * H:(c + 1) * H]
        uv.append(_dot(vc, uu))
        vv.append(_dot(vc, ww))
    vn = jnp.sqrt(vv[0] * vv[0] + vv[1] * vv[1] + vv[2] * vv[2] + 1e-12)
    dotuv = uv[0] * vv[0] + uv[1] * vv[1] + uv[2] * vv[2]
    pre = _dot(s1, w1s_ref[...]) + _dot(vn, w1n_ref[...]) + b1_ref[...]
    a = _dot(_silu(pre), w2_ref[...]) + b2_ref[...]
    s2_ref[...] = s1 + a[:, 0:H] + a[:, H:2 * H] * dotuv
    v2_ref[...] = jnp.concatenate(
        [v1[:, c * H:(c + 1) * H] + a[:, 2 * H:3 * H] * uv[c]
         for c in range(3)], axis=1)


def _update_layer(s, dss, v, dvs, U_, V_, W1s, W1n, b1, W2, b2):
    full = lambda s_: pl.BlockSpec(s_, lambda i: (0, 0))
    return pl.pallas_call(
        _update_body,
        grid=(N // NB,),
        in_specs=[pl.BlockSpec((NB, H), lambda i: (i, 0)),
                  pl.BlockSpec((NB, H), lambda i: (i, 0)),
                  pl.BlockSpec((NB, 3 * H), lambda i: (i, 0)),
                  pl.BlockSpec((NB, 3 * H), lambda i: (i, 0)),
                  full((H, H)), full((H, H)), full((H, H)), full((H, H)),
                  full((1, H)), full((H, 3 * H)), full((1, 3 * H))],
        out_specs=[pl.BlockSpec((NB, H), lambda i: (i, 0)),
                   pl.BlockSpec((NB, 3 * H), lambda i: (i, 0))],
        out_shape=[jax.ShapeDtypeStruct((N, H), jnp.float32),
                   jax.ShapeDtypeStruct((N, 3 * H), jnp.float32)],
    )(s, dss, v, dvs, U_, V_, W1s, W1n, b1, W2, b2)


def _heads_body(s_ref, wc1_ref, bc1_ref, wc2_ref, bc2_ref,
                wy1_ref, by1_ref, wy2_ref, by2_ref, co_ref, ty_ref):
    s = s_ref[...]
    hc = _silu(_dot(s, wc1_ref[...]) + bc1_ref[...])
    co_ref[...] = _dot(hc, wc2_ref[...]) + bc2_ref[...]
    ht = _silu(_dot(s, wy1_ref[...]) + by1_ref[...])
    ty_ref[...] = _dot(ht, wy2_ref[...]) + by2_ref[...]


def _heads(s, Wc1, bc1, Wc2, bc2, Wy1, by1, Wy2, by2):
    full = lambda s_: pl.BlockSpec(s_, lambda i: (0, 0))
    return pl.pallas_call(
        _heads_body,
        grid=(N // NB,),
        in_specs=[pl.BlockSpec((NB, H), lambda i: (i, 0)),
                  full((H, H)), full((1, H)), full((H, 3)), full((1, 3)),
                  full((H, H)), full((1, H)), full((H, A)), full((1, A))],
        out_specs=[pl.BlockSpec((NB, 3), lambda i: (i, 0)),
                   pl.BlockSpec((NB, A), lambda i: (i, 0))],
        out_shape=[jax.ShapeDtypeStruct((N, 3), jnp.float32),
                   jax.ShapeDtypeStruct((N, A), jnp.float32)],
    )(s, Wc1, bc1, Wc2, bc2, Wy1, by1, Wy2, by2)


def kernel(noisy_coords, noisy_types, z, t, edge_index, batch, lattice,
           offsets, atom_embed, Wz, bz, Wt, bt, Wphi1, bphi1, Wphi2, bphi2,
           Wfil, bfil, U, V, Wu1, bu1, Wu2, bu2, Wc1, bc1, Wc2, bc2,
           Wy1, by1, Wy2, by2):
    f32 = jnp.float32
    src = edge_index[0].astype(jnp.int32)
    dst = edge_index[1].astype(jnp.int32)
    types2 = noisy_types.astype(jnp.int32).reshape(N, 1)
    batch2 = batch.astype(jnp.int32).reshape(N, 1)
    t2 = t.astype(jnp.int32).reshape(G, 1)
    latflat = lattice.astype(f32).reshape(G, 9)
    row = lambda b: b.astype(f32).reshape(1, -1)
    zrows = jnp.zeros((FC, H), f32)

    condG = _cond_graphs(z.astype(f32), t2, Wz, row(bz), Wt, row(bt))
    s, cond = _node_init(types2, batch2, atom_embed, condG)
    T = _node_table(noisy_coords.astype(f32), batch2, latflat)
    tsrc, tdst = _gather_pairs(T, src, T, dst)
    geo = _edge_geometry(tsrc, tdst, offsets.astype(f32))

    geo, src = lax.optimization_barrier((geo, src))

    v = jnp.zeros((N, 3 * H), f32)
    for l in range(L):
        s1, phi = _phi_layer(s, cond, Wphi1[l], row(bphi1[l]),
                             Wphi2[l], row(bphi2[l]))
        philist, vlist = _gather_pairs(phi, src, v, src)
        M = _edge_messages(philist, vlist, geo, Wfil[l], row(bfil[l]))
        sums = _scatter_sums(M, dst, zrows)
        s, v = _update_layer(s1, sums[:, 0:H], v, sums[:, H:4 * H],
                             U[l], V[l], Wu1[l][0:H], Wu1[l][H:2 * H],
                             row(bu1[l]), Wu2[l], row(bu2[l]))

    coord, typ = _heads(s, Wc1, row(bc1), Wc2, row(bc2),
                        Wy1, row(by1), Wy2, row(by2))
    return coord, typ


# speedup vs baseline: 17.4755x; 1.1081x over previous
import functools
import math

import jax
import jax.numpy as jnp
from jax import lax
from jax.experimental import pallas as pl
from jax.experimental.pallas import tpu as pltpu
from jax.experimental.pallas import tpu_sc as plsc

N = 10000
E = 160000
H = 128
R = 64
L = 4
LAT = 128
TD = 64
A = 101
G = 64
CUTOFF = 6.0
GAMMA = R / CUTOFF

NB = 1000
EB = 2000
CH = 128
NCHUNK = E // CH
FC = 200
NFCHUNK = N // FC

def _sc_mesh():
    return plsc.VectorSubcoreMesh(core_axis_name="c", subcore_axis_name="s")


def _silu(x):
    return x * jax.nn.sigmoid(x)


def _dot(a, b):
    return jnp.dot(a, b, preferred_element_type=jnp.float32)


PER = E // 32
EC = 40
NC = PER // EC


def _gather_many(pairs):
    npair = len(pairs)
    Ds = [t.shape[1] for t, _ in pairs]
    scratch = ([pltpu.VMEM((PER,), jnp.int32) for _ in range(npair)]
               + [pltpu.VMEM((2 * EC, D), jnp.float32) for D in Ds]
               + [pltpu.SemaphoreType.DMA] * 4)

    @functools.partial(
        pl.kernel,
        mesh=_sc_mesh(),
        out_type=[jax.ShapeDtypeStruct((E, D), jnp.float32) for D in Ds],
        scratch_types=scratch,
    )
    def k(*refs):
        tabs = [refs[2 * i] for i in range(npair)]
        idxs = [refs[2 * i + 1] for i in range(npair)]
        outs = list(refs[2 * npair:3 * npair])
        ixbs = list(refs[3 * npair:4 * npair])
        rings = list(refs[4 * npair:5 * npair])
        semg = list(refs[5 * npair:5 * npair + 2])
        semo = list(refs[5 * npair + 2:5 * npair + 4])

        wid = lax.axis_index("s") * 2 + lax.axis_index("c")
        base = wid * PER
        for i_hbm, ixb in zip(idxs, ixbs):
            pltpu.sync_copy(i_hbm.at[pl.ds(base, PER)], ixb)

        def g_copies(c, b):
            return [pltpu.make_async_copy(
                        t.at[ixb.at[pl.ds(c * EC, EC)]],
                        ring.at[pl.ds(b * EC, EC)], semg[b])
                    for t, ixb, ring in zip(tabs, ixbs, rings)]

        def o_copies(c, b):
            return [pltpu.make_async_copy(
                        ring.at[pl.ds(b * EC, EC)],
                        o.at[pl.ds(base + c * EC, EC)], semo[b])
                    for o, ring in zip(outs, rings)]

        for cp in g_copies(0, 0):
            cp.start()

        @pl.loop(0, (NC + 1) // 2)
        def _(jj):
            for b in (0, 1):
                c = jj * 2 + b

                @pl.when(c < NC)
                def _():
                    @pl.when(c + 1 < NC)
                    def _():
                        @pl.when(c >= 1)
                        def _():
                            for cp in o_copies(c - 1, 1 - b):
                                cp.wait()

                        for cp in g_copies(c + 1, 1 - b):
                            cp.start()

                    for cp in g_copies(c, b):
                        cp.wait()
                    for cp in o_copies(c, b):
                        cp.start()

        for cp in o_copies(NC - 2, (NC - 2) % 2):
            cp.wait()
        for cp in o_copies(NC - 1, (NC - 1) % 2):
            cp.wait()

    flat = []
    for t, i in pairs:
        flat += [t, i]
    out = k(*flat)
    return list(out) if isinstance(out, (list, tuple)) else [out]


def _scatter_sums(M, dst, zrows):
    SPER = E // 16
    SNC = SPER // EC

    @functools.partial(
        pl.kernel,
        mesh=_sc_mesh(),
        out_type=jax.ShapeDtypeStruct((N, 4 * H), jnp.float32),
        scratch_types=[pltpu.VMEM((CH,), jnp.int32),
                       pltpu.VMEM((CH, H), jnp.float32),
                       pltpu.VMEM((FC, H), jnp.float32),
                       pltpu.VMEM_SHARED((N, H), jnp.float32)],
    )
    def k(m_hbm, ix_hbm, z_hbm, out_hbm, idx_v, m_v, fbuf, acc):
        core = lax.axis_index("c")
        sub = lax.axis_index("s")
        nec = NCHUNK // 16 + 1
        nfc = NFCHUNK // 16 + 1

        for p in range(2):
            slab = p * 2 + core

            pltpu.sync_copy(z_hbm, fbuf)

            @pl.loop(0, nfc)
            def _(j):
                fc = j * 16 + sub

                @pl.when(fc < NFCHUNK)
                def _():
                    pltpu.sync_copy(fbuf, acc.at[pl.ds(fc * FC, FC)])

            plsc.subcore_barrier()

            @pl.loop(0, nec)
            def _(j):
                cid = j * 16 + sub

                @pl.when(cid < NCHUNK)
                def _():
                    base = cid * CH
                    pltpu.sync_copy(ix_hbm.at[pl.ds(base, CH)], idx_v)
                    pltpu.sync_copy(
                        m_hbm.at[pl.ds(base, CH), pl.ds(slab * H, H)], m_v)
                    pltpu.sync_copy(m_v, acc.at[idx_v], add=True)

            plsc.subcore_barrier()

            @pl.loop(0, nfc)
            def _(j):
                fc = j * 16 + sub

                @pl.when(fc < NFCHUNK)
                def _():
                    pltpu.sync_copy(acc.at[pl.ds(fc * FC, FC)], fbuf)
                    pltpu.sync_copy(
                        fbuf, out_hbm.at[pl.ds(fc * FC, FC),
                                         pl.ds(slab * H, H)])

            plsc.subcore_barrier()

    return k(M, dst, zrows)


def _cond_body(z_ref, t_ref, wz_ref, bz_ref, wt_ref, bt_ref, o_ref):
    t = t_ref[...].astype(jnp.float32)
    half = TD // 2
    j = lax.broadcasted_iota(jnp.int32, (1, half), 1).astype(jnp.float32)
    freqs = jnp.exp(-math.log(10000.0) * j / (half - 1))
    emb = t * freqs
    temb = jnp.concatenate([jnp.sin(emb), jnp.cos(emb)], axis=1)
    o_ref[...] = (_dot(z_ref[...], wz_ref[...]) + bz_ref[...]
                  + _dot(temb, wt_ref[...]) + bt_ref[...])


def _cond_graphs(z, t, Wz, bz, Wt, bt):
    return pl.pallas_call(
        _cond_body,
        out_shape=jax.ShapeDtypeStruct((G, H), jnp.float32),
    )(z, t, Wz, bz, Wt, bt)


def _init_body(ty_ref, b_ref, ae_ref, cg_ref, s0_ref, cond_ref):
    ty = ty_ref[...]
    oh = (ty == lax.broadcasted_iota(jnp.int32, (NB, A), 1))
    s0_ref[...] = _dot(oh.astype(jnp.float32), ae_ref[...])
    b = b_ref[...]
    ohb = (b == lax.broadcasted_iota(jnp.int32, (NB, G), 1))
    cond_ref[...] = _dot(ohb.astype(jnp.float32), cg_ref[...])


def _node_init(types2, batch2, atom_embed, condG):
    full = lambda s: pl.BlockSpec(s, lambda i: (0, 0))
    return pl.pallas_call(
        _init_body,
        grid=(N // NB,),
        in_specs=[pl.BlockSpec((NB, 1), lambda i: (i, 0)),
                  pl.BlockSpec((NB, 1), lambda i: (i, 0)),
                  full((A, H)), full((G, H))],
        out_specs=[pl.BlockSpec((NB, H), lambda i: (i, 0)),
                   pl.BlockSpec((NB, H), lambda i: (i, 0))],
        out_shape=[jax.ShapeDtypeStruct((N, H), jnp.float32),
                   jax.ShapeDtypeStruct((N, H), jnp.float32)],
    )(types2, batch2, atom_embed, condG)


def _table_body(xy_ref, b_ref, lat_ref, t_ref):
    b = b_ref[...]
    ohb = (b == lax.broadcasted_iota(jnp.int32, (NB, G), 1))
    lrows = _dot(ohb.astype(jnp.float32), lat_ref[...])
    t_ref[...] = jnp.concatenate(
        [xy_ref[...], lrows, jnp.zeros((NB, 116), jnp.float32)], axis=1)


def _node_table(coords, batch2, latflat):
    return pl.pallas_call(
        _table_body,
        grid=(N // NB,),
        in_specs=[pl.BlockSpec((NB, 3), lambda i: (i, 0)),
                  pl.BlockSpec((NB, 1), lambda i: (i, 0)),
                  pl.BlockSpec((G, 9), lambda i: (0, 0))],
        out_specs=pl.BlockSpec((NB, 128), lambda i: (i, 0)),
        out_shape=jax.ShapeDtypeStruct((N, 128), jnp.float32),
    )(coords, batch2, latflat)


def _geo_body(ts_ref, td_ref, off_ref, geo_ref):
    ts = ts_ref[...]
    td = td_ref[...]
    frac = td[:, 0:3] - ts[:, 0:3] + off_ref[...]
    lf = ts[:, 3:12]
    cols = []
    for jx in range(3):
        cj = (frac[:, 0:1] * lf[:, jx:jx + 1]
              + frac[:, 1:2] * lf[:, 3 + jx:4 + jx]
              + frac[:, 2:3] * lf[:, 6 + jx:7 + jx])
        cols.append(cj)
    cart = jnp.concatenate(cols, axis=1)
    nrm = jnp.sqrt(jnp.sum(cart * cart, axis=1, keepdims=True))
    d = jnp.clip(nrm, 1e-8, None)
    unit = cart / jnp.maximum(nrm, 1e-12)
    geo_ref[...] = jnp.concatenate(
        [d, unit, jnp.zeros((EB, 4), jnp.float32)], axis=1)


def _edge_geometry(tsrc, tdst, offsets):
    return pl.pallas_call(
        _geo_body,
        grid=(E // EB,),
        in_specs=[pl.BlockSpec((EB, 128), lambda i: (i, 0)),
                  pl.BlockSpec((EB, 128), lambda i: (i, 0)),
                  pl.BlockSpec((EB, 3), lambda i: (i, 0))],
        out_specs=pl.BlockSpec((EB, 8), lambda i: (i, 0)),
        out_shape=jax.ShapeDtypeStruct((E, 8), jnp.float32),
    )(tsrc, tdst, offsets)


def _phi_body(s_ref, c_ref, w1_ref, b1_ref, w2_ref, b2_ref,
              snew_ref, phi_ref):
    s1 = s_ref[...] + c_ref[...]
    snew_ref[...] = s1
    h = _silu(_dot(s1, w1_ref[...]) + b1_ref[...])
    phi_ref[...] = _dot(h, w2_ref[...]) + b2_ref[...]


def _phi_layer(s, cond, W1, b1, W2, b2):
    full = lambda s_: pl.BlockSpec(s_, lambda i: (0, 0))
    return pl.pallas_call(
        _phi_body,
        grid=(N // NB,),
        in_specs=[pl.BlockSpec((NB, H), lambda i: (i, 0)),
                  pl.BlockSpec((NB, H), lambda i: (i, 0)),
                  full((H, H)), full((1, H)), full((H, 3 * H)),
                  full((1, 3 * H))],
        out_specs=[pl.BlockSpec((NB, H), lambda i: (i, 0)),
                   pl.BlockSpec((NB, 3 * H), lambda i: (i, 0))],
        out_shape=[jax.ShapeDtypeStruct((N, H), jnp.float32),
                   jax.ShapeDtypeStruct((N, 3 * H), jnp.float32)],
    )(s, cond, W1, b1, W2, b2)


def _edge_body(pl_ref, vl_ref, geo_ref, wf_ref, bf_ref, m_ref):
    g = geo_ref[...]
    d = g[:, 0:1]
    centers = lax.broadcasted_iota(jnp.int32, (1, R), 1).astype(
        jnp.float32) * (CUTOFF / (R - 1))
    erbf = jnp.exp(-GAMMA * (d - centers) ** 2)
    filt = _dot(erbf, wf_ref[...]) + bf_ref[...]
    m = pl_ref[...] * filt
    ds = m[:, 0:H]
    dvg = m[:, H:2 * H]
    dvd = m[:, 2 * H:3 * H]
    vl = vl_ref[...]
    parts = [ds]
    for c in range(3):
        parts.append(dvg * vl[:, c * H:(c + 1) * H] + dvd * g[:, 1 + c:2 + c])
    m_ref[...] = jnp.concatenate(parts, axis=1)


def _edge_body0(pl_ref, geo_ref, wf_ref, bf_ref, m_ref):
    g = geo_ref[...]
    d = g[:, 0:1]
    centers = lax.broadcasted_iota(jnp.int32, (1, R), 1).astype(
        jnp.float32) * (CUTOFF / (R - 1))
    erbf = jnp.exp(-GAMMA * (d - centers) ** 2)
    filt = _dot(erbf, wf_ref[...]) + bf_ref[...]
    m = pl_ref[...] * filt
    ds = m[:, 0:H]
    dvd = m[:, 2 * H:3 * H]
    parts = [ds] + [dvd * g[:, 1 + c:2 + c] for c in range(3)]
    m_ref[...] = jnp.concatenate(parts, axis=1)


def _edge_messages(philist, vlist, geo, Wf, bf):
    full = lambda s_: pl.BlockSpec(s_, lambda i: (0, 0))
    espec = pl.BlockSpec((EB, 3 * H), lambda i: (i, 0))
    ins = [philist] + ([vlist] if vlist is not None else []) + [geo, Wf, bf]
    return pl.pallas_call(
        _edge_body if vlist is not None else _edge_body0,
        grid=(E // EB,),
        in_specs=[espec] + ([espec] if vlist is not None else [])
                 + [pl.BlockSpec((EB, 8), lambda i: (i, 0)),
                    full((R, 3 * H)), full((1, 3 * H))],
        out_specs=pl.BlockSpec((EB, 4 * H), lambda i: (i, 0)),
        out_shape=jax.ShapeDtypeStruct((E, 4 * H), jnp.float32),
    )(*ins)


def _update_body(s_ref, dss_ref, v_ref, dvs_ref, u_ref, vv_ref,
                 w1s_ref, w1n_ref, b1_ref, w2_ref, b2_ref,
                 s2_ref, v2_ref):
    s1 = s_ref[...] + dss_ref[...]
    v1 = v_ref[...] + dvs_ref[...]
    uu = u_ref[...]
    ww = vv_ref[...]
    uv = []
    vv = []
    for c in range(3):
        vc = v1[:, c * H:(c + 1) * H]
        uv.append(_dot(vc, uu))
        vv.append(_dot(vc, ww))
    vn = jnp.sqrt(vv[0] * vv[0] + vv[1] * vv[1] + vv[2] * vv[2] + 1e-12)
    dotuv = uv[0] * vv[0] + uv[1] * vv[1] + uv[2] * vv[2]
    pre = _dot(s1, w1s_ref[...]) + _dot(vn, w1n_ref[...]) + b1_ref[...]
    a = _dot(_silu(pre), w2_ref[...]) + b2_ref[...]
    s2_ref[...] = s1 + a[:, 0:H] + a[:, H:2 * H] * dotuv
    v2_ref[...] = jnp.concatenate(
        [v1[:, c * H:(c + 1) * H] + a[:, 2 * H:3 * H] * uv[c]
         for c in range(3)], axis=1)


def _update_layer(s, dss, v, dvs, U_, V_, W1s, W1n, b1, W2, b2):
    full = lambda s_: pl.BlockSpec(s_, lambda i: (0, 0))
    return pl.pallas_call(
        _update_body,
        grid=(N // NB,),
        in_specs=[pl.BlockSpec((NB, H), lambda i: (i, 0)),
                  pl.BlockSpec((NB, H), lambda i: (i, 0)),
                  pl.BlockSpec((NB, 3 * H), lambda i: (i, 0)),
                  pl.BlockSpec((NB, 3 * H), lambda i: (i, 0)),
                  full((H, H)), full((H, H)), full((H, H)), full((H, H)),
                  full((1, H)), full((H, 3 * H)), full((1, 3 * H))],
        out_specs=[pl.BlockSpec((NB, H), lambda i: (i, 0)),
                   pl.BlockSpec((NB, 3 * H), lambda i: (i, 0))],
        out_shape=[jax.ShapeDtypeStruct((N, H), jnp.float32),
                   jax.ShapeDtypeStruct((N, 3 * H), jnp.float32)],
    )(s, dss, v, dvs, U_, V_, W1s, W1n, b1, W2, b2)


def _heads_body(s_ref, wc1_ref, bc1_ref, wc2_ref, bc2_ref,
                wy1_ref, by1_ref, wy2_ref, by2_ref, co_ref, ty_ref):
    s = s_ref[...]
    hc = _silu(_dot(s, wc1_ref[...]) + bc1_ref[...])
    co_ref[...] = _dot(hc, wc2_ref[...]) + bc2_ref[...]
    ht = _silu(_dot(s, wy1_ref[...]) + by1_ref[...])
    ty_ref[...] = _dot(ht, wy2_ref[...]) + by2_ref[...]


def _heads(s, Wc1, bc1, Wc2, bc2, Wy1, by1, Wy2, by2):
    full = lambda s_: pl.BlockSpec(s_, lambda i: (0, 0))
    return pl.pallas_call(
        _heads_body,
        grid=(N // NB,),
        in_specs=[pl.BlockSpec((NB, H), lambda i: (i, 0)),
                  full((H, H)), full((1, H)), full((H, 3)), full((1, 3)),
                  full((H, H)), full((1, H)), full((H, A)), full((1, A))],
        out_specs=[pl.BlockSpec((NB, 3), lambda i: (i, 0)),
                   pl.BlockSpec((NB, A), lambda i: (i, 0))],
        out_shape=[jax.ShapeDtypeStruct((N, 3), jnp.float32),
                   jax.ShapeDtypeStruct((N, A), jnp.float32)],
    )(s, Wc1, bc1, Wc2, bc2, Wy1, by1, Wy2, by2)


def kernel(noisy_coords, noisy_types, z, t, edge_index, batch, lattice,
           offsets, atom_embed, Wz, bz, Wt, bt, Wphi1, bphi1, Wphi2, bphi2,
           Wfil, bfil, U, V, Wu1, bu1, Wu2, bu2, Wc1, bc1, Wc2, bc2,
           Wy1, by1, Wy2, by2):
    f32 = jnp.float32
    src = edge_index[0].astype(jnp.int32)
    dst = edge_index[1].astype(jnp.int32)
    types2 = noisy_types.astype(jnp.int32).reshape(N, 1)
    batch2 = batch.astype(jnp.int32).reshape(N, 1)
    t2 = t.astype(jnp.int32).reshape(G, 1)
    latflat = lattice.astype(f32).reshape(G, 9)
    row = lambda b: b.astype(f32).reshape(1, -1)
    zrows = jnp.zeros((FC, H), f32)

    condG = _cond_graphs(z.astype(f32), t2, Wz, row(bz), Wt, row(bt))
    s, cond = _node_init(types2, batch2, atom_embed, condG)
    T = _node_table(noisy_coords.astype(f32), batch2, latflat)
    tsrc, tdst = _gather_many([(T, src), (T, dst)])
    geo = _edge_geometry(tsrc, tdst, offsets.astype(f32))

    geo, src = lax.optimization_barrier((geo, src))

    v = jnp.zeros((N, 3 * H), f32)
    for l in range(L):
        s1, phi = _phi_layer(s, cond, Wphi1[l], row(bphi1[l]),
                             Wphi2[l], row(bphi2[l]))
        if l == 0:
            (philist,) = _gather_many([(phi, src)])
            vlist = None
        else:
            philist, vlist = _gather_many([(phi, src), (v, src)])
        M = _edge_messages(philist, vlist, geo, Wfil[l], row(bfil[l]))
        sums = _scatter_sums(M, dst, zrows)
        s, v = _update_layer(s1, sums[:, 0:H], v, sums[:, H:4 * H],
                             U[l], V[l], Wu1[l][0:H], Wu1[l][H:2 * H],
                             row(bu1[l]), Wu2[l], row(bu2[l]))

    coord, typ = _heads(s, Wc1, row(bc1), Wc2, row(bc2),
                        Wy1, row(by1), Wy2, row(by2))
    return coord, typ
